# baseline (speedup 1.0000x reference)
import functools

import jax
import jax.numpy as jnp
from jax.experimental import pallas as pl

S, H, D, T = 512, 32, 128, 8192
BS = 128
CT = 256


def _indexer_kernel(q_ref, k_ref, ks_ref, ke_ref, out_ref):
    qbf = q_ref[...].reshape(H * BS, D)
    ks = ks_ref[...]
    ke = ke_ref[...]
    for c in range(T // CT):
        scores = jax.lax.dot_general(
            qbf, k_ref[c * CT:(c + 1) * CT, :],
            dimension_numbers=(((1,), (1,)), ((), ())),
            preferred_element_type=jnp.float32,
        )
        scores = jnp.maximum(scores, 0.0)
        logits = scores.reshape(H, BS, CT).sum(axis=0)
        t_idx = c * CT + jax.lax.broadcasted_iota(jnp.int32, (BS, CT), 1)
        mask = (t_idx >= ks) & (t_idx < ke)
        out_ref[:, c * CT:(c + 1) * CT] = jnp.where(mask, logits, -jnp.inf)


@functools.partial(jax.jit, static_argnames=())
def kernel(index_q, index_k, weights, cu_seqlen_ks, cu_seqlen_ke):
    q3 = (index_q * weights[:, :, None]).astype(jnp.bfloat16).transpose(1, 0, 2)
    kbf = index_k.astype(jnp.bfloat16)
    ks2 = cu_seqlen_ks.reshape(S, 1)
    ke2 = cu_seqlen_ke.reshape(S, 1)

    out = pl.pallas_call(
        _indexer_kernel,
        grid=(S // BS,),
        in_specs=[
            pl.BlockSpec((H, BS, D), lambda si: (0, si, 0)),
            pl.BlockSpec((T, D), lambda si: (0, 0)),
            pl.BlockSpec((BS, 1), lambda si: (si, 0)),
            pl.BlockSpec((BS, 1), lambda si: (si, 0)),
        ],
        out_specs=pl.BlockSpec((BS, T), lambda si: (si, 0)),
        out_shape=jax.ShapeDtypeStruct((S, T), jnp.float32),
    )(q3, kbf, ks2, ke2)
    return out
